# baseline (device time: 45129 ns/iter reference)
import jax
import jax.numpy as jnp
from jax import lax
from jax.experimental import pallas as pl
from jax.experimental.pallas import tpu as pltpu

N_DEV = 8
N_HOP = 3
N_PC = 8


def kernel(x, w_mat):
    m_per, k = x.shape
    _, n_per = w_mat.shape
    half = m_per // N_PC

    def body(x_ref, w_ref, out_ref, gathered, w_bf, cw_send, cw_recv,
             ccw_send, ccw_recv, ch_send, ch_recv):
        me = lax.axis_index("i")

        def g(r):
            return jnp.where(r < 4, r, 11 - r)

        ri = g(me)
        right_log = g(lax.rem(ri + 1, N_DEV))
        left_log = g(lax.rem(ri + 7, N_DEV))
        even = lax.rem(ri, 2) == 0
        partner_log = g(lax.rem(jnp.where(even, ri + 3, ri + 5), N_DEV))
        fwd_log = g(lax.rem(jnp.where(even, ri + 7, ri + 1), N_DEV))

        barrier_sem = pltpu.get_barrier_semaphore()
        for nbr in (left_log, right_log, partner_log):
            pl.semaphore_signal(
                barrier_sem, inc=1,
                device_id=(nbr,), device_id_type=pl.DeviceIdType.MESH,
            )
        pl.semaphore_wait(barrier_sem, 3)

        my_row0 = me * m_per

        def mk_ring(direction, h, p):
            if direction == "cw":
                o = g(lax.rem(ri - h + N_DEV, N_DEV))
                dst_dev, sends, recvs = right_log, cw_send, cw_recv
            else:
                o = g(lax.rem(ri + h, N_DEV))
                dst_dev, sends, recvs = left_log, ccw_send, ccw_recv
            sl = pl.ds(o * m_per + p * half, half)
            idx = h * N_PC + p
            return pltpu.make_async_remote_copy(
                src_ref=gathered.at[sl, :],
                dst_ref=gathered.at[sl, :],
                send_sem=sends.at[idx],
                recv_sem=recvs.at[idx],
                device_id=(dst_dev,),
                device_id_type=pl.DeviceIdType.MESH,
            )

        rd = {}
        for p in range(N_PC):
            psl = pl.ds(p * half, half)
            gathered[pl.ds(my_row0 + p * half, half), :] = (
                x_ref[psl, :].astype(jnp.bfloat16)
            )
            rd["cw", 0, p] = mk_ring("cw", 0, p)
            rd["cw", 0, p].start()
            rd["ccw", 0, p] = mk_ring("ccw", 0, p)
            rd["ccw", 0, p].start()

        w_bf[:, :] = w_ref[:, :].astype(jnp.bfloat16)

        def gemm(row_start, rows):
            out_ref[pl.ds(row_start, rows), :] = jnp.maximum(
                jnp.dot(
                    gathered[pl.ds(row_start, rows), :],
                    w_bf[:, :],
                    preferred_element_type=jnp.float32,
                ),
                0.0,
            )

        gemm(my_row0, m_per)

        for h in range(N_HOP - 1):
            for p in range(N_PC):
                rd["cw", h, p].wait_recv()
                rd["cw", h + 1, p] = mk_ring("cw", h + 1, p)
                rd["cw", h + 1, p].start()
                rd["ccw", h, p].wait_recv()
                rd["ccw", h + 1, p] = mk_ring("ccw", h + 1, p)
                rd["ccw", h + 1, p].start()
            if h == 0:
                ch_sl = pl.ds(fwd_log * m_per, m_per)
                chord = pltpu.make_async_remote_copy(
                    src_ref=gathered.at[ch_sl, :],
                    dst_ref=gathered.at[ch_sl, :],
                    send_sem=ch_send.at[0],
                    recv_sem=ch_recv.at[0],
                    device_id=(partner_log,),
                    device_id_type=pl.DeviceIdType.MESH,
                )
                chord.start()
            gemm(g(lax.rem(ri + 7 - h, N_DEV)) * m_per, m_per)
            gemm(g(lax.rem(ri + 1 + h, N_DEV)) * m_per, m_per)

        chord.wait_recv()
        gemm(g(lax.rem(ri + 4, N_DEV)) * m_per, m_per)

        o_cw = g(lax.rem(ri + 5, N_DEV))
        o_ccw = g(lax.rem(ri + 3, N_DEV))
        for p in range(N_PC):
            rd["cw", 2, p].wait_recv()
            gemm(o_cw * m_per + p * half, half)
            rd["ccw", 2, p].wait_recv()
            gemm(o_ccw * m_per + p * half, half)

        for h in range(N_HOP):
            for p in range(N_PC):
                rd["cw", h, p].wait_send()
                rd["ccw", h, p].wait_send()
        chord.wait_send()

    return pl.pallas_call(
        body,
        out_shape=jax.ShapeDtypeStruct((N_DEV * m_per, n_per), jnp.float32),
        in_specs=[
            pl.BlockSpec(memory_space=pltpu.VMEM),
            pl.BlockSpec(memory_space=pltpu.VMEM),
        ],
        out_specs=pl.BlockSpec(memory_space=pltpu.VMEM),
        scratch_shapes=[
            pltpu.VMEM((N_DEV * m_per, k), jnp.bfloat16),
            pltpu.VMEM((k, n_per), jnp.bfloat16),
            pltpu.SemaphoreType.DMA((N_HOP * N_PC,)),
            pltpu.SemaphoreType.DMA((N_HOP * N_PC,)),
            pltpu.SemaphoreType.DMA((N_HOP * N_PC,)),
            pltpu.SemaphoreType.DMA((N_HOP * N_PC,)),
            pltpu.SemaphoreType.DMA((1,)),
            pltpu.SemaphoreType.DMA((1,)),
        ],
        compiler_params=pltpu.CompilerParams(collective_id=0),
    )(x, w_mat)


# device time: 39083 ns/iter; 1.1547x vs baseline; 1.1547x over previous
import jax
import jax.numpy as jnp
from jax import lax
from jax.experimental import pallas as pl
from jax.experimental.pallas import tpu as pltpu

N_DEV = 8
N_PC = 4
N_ST = 10


def kernel(x, w_mat):
    m_per, k = x.shape
    _, n_per = w_mat.shape
    pc = m_per // N_PC
    half = m_per // 2

    def body(x_ref, w_ref, out_ref, gathered, w_bf, cw_s, cw_r,
             ccw_s, ccw_r, ch_s, ch_r):
        me = lax.axis_index("i")

        def g(r):
            return jnp.where(r < 4, r, 11 - r)

        def rmod(r):
            return lax.rem(r + 2 * N_DEV, N_DEV)

        ri = g(me)
        right_log = g(rmod(ri + 1))
        left_log = g(rmod(ri - 1))
        even = lax.rem(ri, 2) == 0
        partner_log = g(rmod(jnp.where(even, ri + 3, ri - 3)))

        def W(a, b):
            return jnp.where(even, a, b)

        barrier_sem = pltpu.get_barrier_semaphore()
        for nbr in (left_log, right_log, partner_log):
            pl.semaphore_signal(
                barrier_sem, inc=1,
                device_id=(nbr,), device_id_type=pl.DeviceIdType.MESH,
            )
        pl.semaphore_wait(barrier_sem, 3)

        def rows(c_ring, piece, npc=1):
            return pl.ds(g(rmod(c_ring)) * m_per + piece * pc, npc * pc)

        def mk(sl, sems_s, sems_r, idx, dev):
            return pltpu.make_async_remote_copy(
                src_ref=gathered.at[sl, :],
                dst_ref=gathered.at[sl, :],
                send_sem=sems_s.at[idx],
                recv_sem=sems_r.at[idx],
                device_id=(dev,),
                device_id_type=pl.DeviceIdType.MESH,
            )

        def cw_desc(j):
            if j < 4:
                c, p = ri, j
            elif j < 6:
                c, p = rmod(ri - 1), j - 4
            elif j < 8:
                c, p = rmod(W(ri - 1, ri - 2)), W(j - 4, j - 6)
            else:
                c, p = rmod(W(ri + 3, ri - 2)), j - 6
            return mk(rows(c, p), cw_s, cw_r, j, right_log)

        def ccw_desc(i):
            if i < 4:
                c, p = ri, i
            elif i < 6:
                c, p = rmod(ri + 1), i - 4
            elif i < 8:
                c, p = rmod(W(ri + 2, ri + 1)), W(i - 6, i - 4)
            else:
                c, p = rmod(W(ri + 2, ri - 3)), i - 6
            return mk(rows(c, p), ccw_s, ccw_r, i, left_log)

        rcw = [cw_desc(j) for j in range(N_ST)]
        rccw = [ccw_desc(i) for i in range(N_ST)]
        ch0 = mk(rows(ri, 0, N_PC), ch_s, ch_r, 0, partner_log)
        ch1 = mk(rows(W(ri - 1, ri + 1), 0, N_PC), ch_s, ch_r, 1, partner_log)

        for p in range(N_PC):
            gathered[pl.ds(me * m_per + p * pc, pc), :] = (
                x_ref[pl.ds(p * pc, pc), :].astype(jnp.bfloat16)
            )
            rcw[p].start()
            rccw[p].start()
        ch0.start()

        w_bf[:, :] = w_ref[:, :].astype(jnp.bfloat16)

        def gemm(c_ring, piece=0, npc=N_PC):
            sl = rows(c_ring, piece, npc)
            out_ref[sl, :] = jnp.maximum(
                jnp.dot(
                    gathered[sl, :],
                    w_bf[:, :],
                    preferred_element_type=jnp.float32,
                ),
                0.0,
            )

        gemm(ri)

        for kk in range(4):
            rcw[kk].wait_recv()
            if kk < 2:
                rcw[4 + kk].start()
            else:
                @pl.when(even)
                def _():
                    rcw[4 + kk].start()
            rccw[kk].wait_recv()
            if kk < 2:
                rccw[4 + kk].start()
            else:
                @pl.when(~even)
                def _():
                    rccw[4 + kk].start()

        ch1.start()
        gemm(rmod(ri - 1))
        gemm(rmod(ri + 1))

        ch0.wait_recv()

        @pl.when(even)
        def _():
            rcw[8].start()
            rcw[9].start()

        @pl.when(~even)
        def _():
            rccw[8].start()
            rccw[9].start()

        gemm(rmod(W(ri + 3, ri - 3)))

        for kk in range(4, 8):
            rcw[kk].wait_recv()

            @pl.when(~even)
            def _():
                rcw[kk + 2].start()

            rccw[kk].wait_recv()

            @pl.when(even)
            def _():
                rccw[kk + 2].start()

            if kk == 5:
                gemm(rmod(W(ri - 2, ri + 2)), 0, 2)

        gemm(rmod(W(ri + 2, ri - 2)))
        gemm(rmod(W(ri - 3, ri + 3)), 0, 2)

        ch1.wait_recv()
        gemm(rmod(ri + 4))

        rcw[8].wait_recv()
        rccw[8].wait_recv()
        rcw[9].wait_recv()
        rccw[9].wait_recv()
        gemm(rmod(W(ri - 3, ri + 3)), 2, 2)
        gemm(rmod(W(ri - 2, ri + 2)), 2, 2)

        for d in rcw + rccw + [ch0, ch1]:
            d.wait_send()

    return pl.pallas_call(
        body,
        out_shape=jax.ShapeDtypeStruct((N_DEV * m_per, n_per), jnp.float32),
        in_specs=[
            pl.BlockSpec(memory_space=pltpu.VMEM),
            pl.BlockSpec(memory_space=pltpu.VMEM),
        ],
        out_specs=pl.BlockSpec(memory_space=pltpu.VMEM),
        scratch_shapes=[
            pltpu.VMEM((N_DEV * m_per, k), jnp.bfloat16),
            pltpu.VMEM((k, n_per), jnp.bfloat16),
            pltpu.SemaphoreType.DMA((N_ST,)),
            pltpu.SemaphoreType.DMA((N_ST,)),
            pltpu.SemaphoreType.DMA((N_ST,)),
            pltpu.SemaphoreType.DMA((N_ST,)),
            pltpu.SemaphoreType.DMA((2,)),
            pltpu.SemaphoreType.DMA((2,)),
        ],
        compiler_params=pltpu.CompilerParams(collective_id=0),
    )(x, w_mat)


# device time: 38786 ns/iter; 1.1635x vs baseline; 1.0077x over previous
import jax
import jax.numpy as jnp
from jax import lax
from jax.experimental import pallas as pl
from jax.experimental.pallas import tpu as pltpu

N_DEV = 8
N_PC = 4
N_ST = 10


def kernel(x, w_mat):
    m_per, k = x.shape
    _, n_per = w_mat.shape
    pc = m_per // N_PC
    half = m_per // 2

    def body(x_ref, w_ref, out_ref, gathered, w_bf, cw_s, cw_r,
             ccw_s, ccw_r, ch_s, ch_r):
        me = lax.axis_index("i")

        def g(r):
            return jnp.where(r < 4, r, 11 - r)

        def rmod(r):
            return lax.rem(r + 2 * N_DEV, N_DEV)

        ri = g(me)
        right_log = g(rmod(ri + 1))
        left_log = g(rmod(ri - 1))
        even = lax.rem(ri, 2) == 0
        partner_log = g(rmod(jnp.where(even, ri + 3, ri - 3)))

        def W(a, b):
            return jnp.where(even, a, b)

        barrier_sem = pltpu.get_barrier_semaphore()
        for nbr in (left_log, right_log, partner_log):
            pl.semaphore_signal(
                barrier_sem, inc=1,
                device_id=(nbr,), device_id_type=pl.DeviceIdType.MESH,
            )
        pl.semaphore_wait(barrier_sem, 3)

        def rows(c_ring, piece, npc=1):
            return pl.ds(g(rmod(c_ring)) * m_per + piece * pc, npc * pc)

        def mk(sl, sems_s, sems_r, idx, dev):
            return pltpu.make_async_remote_copy(
                src_ref=gathered.at[sl, :],
                dst_ref=gathered.at[sl, :],
                send_sem=sems_s.at[idx],
                recv_sem=sems_r.at[idx],
                device_id=(dev,),
                device_id_type=pl.DeviceIdType.MESH,
            )

        def cw_desc(j):
            if j < 4:
                c, p = ri, j
            elif j < 6:
                c, p = rmod(ri - 1), j - 4
            elif j < 8:
                c, p = rmod(W(ri - 1, ri - 2)), W(j - 4, j - 6)
            else:
                c, p = rmod(W(ri + 3, ri - 2)), j - 6
            return mk(rows(c, p), cw_s, cw_r, j, right_log)

        def ccw_desc(i):
            if i < 4:
                c, p = ri, i
            elif i < 6:
                c, p = rmod(ri + 1), i - 4
            elif i < 8:
                c, p = rmod(W(ri + 2, ri + 1)), W(i - 6, i - 4)
            else:
                c, p = rmod(W(ri + 2, ri - 3)), i - 6
            return mk(rows(c, p), ccw_s, ccw_r, i, left_log)

        rcw = [cw_desc(j) for j in range(N_ST)]
        rccw = [ccw_desc(i) for i in range(N_ST)]
        ch0 = mk(rows(ri, 0, N_PC), ch_s, ch_r, 0, partner_log)
        ch1 = mk(rows(W(ri - 1, ri + 1), 0, N_PC), ch_s, ch_r, 1, partner_log)

        for p in range(N_PC):
            gathered[pl.ds(me * m_per + p * pc, pc), :] = (
                x_ref[pl.ds(p * pc, pc), :].astype(jnp.bfloat16)
            )
            rcw[p].start()
            rccw[p].start()
        ch0.start()

        w_bf[:, :] = w_ref[:, :].astype(jnp.bfloat16)

        def gemm(c_ring, piece=0, npc=N_PC):
            sl = rows(c_ring, piece, npc)
            out_ref[sl, :] = jnp.maximum(
                jnp.dot(
                    gathered[sl, :],
                    w_bf[:, :],
                    preferred_element_type=jnp.float32,
                ),
                0.0,
            )

        gemm(ri)

        for kk in range(4):
            rcw[kk].wait_recv()
            if kk < 2:
                rcw[4 + kk].start()
            else:
                @pl.when(even)
                def _():
                    rcw[4 + kk].start()
            rccw[kk].wait_recv()
            if kk < 2:
                rccw[4 + kk].start()
            else:
                @pl.when(~even)
                def _():
                    rccw[4 + kk].start()

        ch1.start()
        gemm(rmod(ri - 1), 0, 2)

        ch0.wait_recv()

        @pl.when(even)
        def _():
            rcw[8].start()
            rcw[9].start()

        @pl.when(~even)
        def _():
            rccw[8].start()
            rccw[9].start()

        gemm(rmod(ri - 1), 2, 2)
        gemm(rmod(ri + 1), 0, 2)
        gemm(rmod(ri + 1), 2, 2)

        for kk in range(4, 8):
            rcw[kk].wait_recv()

            @pl.when(~even)
            def _():
                rcw[kk + 2].start()

            rccw[kk].wait_recv()

            @pl.when(even)
            def _():
                rccw[kk + 2].start()

            if kk == 4:
                gemm(rmod(W(ri + 3, ri - 3)), 0, 2)
            elif kk == 5:
                gemm(rmod(W(ri + 3, ri - 3)), 2, 2)
                gemm(rmod(W(ri - 2, ri + 2)), 0, 2)
            elif kk == 6:
                gemm(rmod(W(ri + 2, ri - 2)), 0, 2)
            else:
                gemm(rmod(W(ri + 2, ri - 2)), 2, 2)
                gemm(rmod(W(ri - 3, ri + 3)), 0, 2)

        ch1.wait_recv()
        gemm(rmod(ri + 4))

        rcw[8].wait_recv()
        rccw[8].wait_recv()
        gemm(rmod(W(ri - 3, ri + 3)), 2, 1)
        gemm(rmod(W(ri - 2, ri + 2)), 2, 1)
        rcw[9].wait_recv()
        rccw[9].wait_recv()
        gemm(rmod(W(ri - 3, ri + 3)), 3, 1)
        gemm(rmod(W(ri - 2, ri + 2)), 3, 1)

        for d in rcw + rccw + [ch0, ch1]:
            d.wait_send()

    return pl.pallas_call(
        body,
        out_shape=jax.ShapeDtypeStruct((N_DEV * m_per, n_per), jnp.float32),
        in_specs=[
            pl.BlockSpec(memory_space=pltpu.VMEM),
            pl.BlockSpec(memory_space=pltpu.VMEM),
        ],
        out_specs=pl.BlockSpec(memory_space=pltpu.VMEM),
        scratch_shapes=[
            pltpu.VMEM((N_DEV * m_per, k), jnp.bfloat16),
            pltpu.VMEM((k, n_per), jnp.bfloat16),
            pltpu.SemaphoreType.DMA((N_ST,)),
            pltpu.SemaphoreType.DMA((N_ST,)),
            pltpu.SemaphoreType.DMA((N_ST,)),
            pltpu.SemaphoreType.DMA((N_ST,)),
            pltpu.SemaphoreType.DMA((2,)),
            pltpu.SemaphoreType.DMA((2,)),
        ],
        compiler_params=pltpu.CompilerParams(collective_id=0),
    )(x, w_mat)
